# baseline (speedup 1.0000x reference)
import jax
import jax.numpy as jnp
from jax.experimental import pallas as pl
from jax.experimental.pallas import tpu as pltpu


def _copy_dma(w_ref, o_ref, sem):
    pltpu.make_async_copy(w_ref, o_ref, sem).start()
    pltpu.make_async_copy(w_ref, o_ref, sem).wait()


def kernel(x, W):
    del x
    S, D = W.shape
    return pl.pallas_call(
        _copy_dma,
        in_specs=[pl.BlockSpec(memory_space=pl.ANY)],
        out_specs=pl.BlockSpec(memory_space=pl.ANY),
        out_shape=jax.ShapeDtypeStruct((S, D), W.dtype),
        scratch_shapes=[pltpu.SemaphoreType.DMA],
    )(W)


# speedup vs baseline: 42.8359x; 42.8359x over previous
import jax
import jax.numpy as jnp
from jax.experimental import pallas as pl
from jax.experimental.pallas import tpu as pltpu


def _copy_block(w_ref, o_ref):
    o_ref[...] = w_ref[...]


def kernel(x, W):
    del x
    S, D = W.shape
    blk = 1024
    return pl.pallas_call(
        _copy_block,
        grid=(S // blk,),
        in_specs=[pl.BlockSpec((blk, D), lambda i: (i, 0))],
        out_specs=pl.BlockSpec((blk, D), lambda i: (i, 0)),
        out_shape=jax.ShapeDtypeStruct((S, D), W.dtype),
        compiler_params=pltpu.CompilerParams(
            dimension_semantics=("parallel",),
        ),
    )(W)


# speedup vs baseline: 45.9162x; 1.0719x over previous
import jax
import jax.numpy as jnp
from jax.experimental import pallas as pl
from jax.experimental.pallas import tpu as pltpu


def _copy_block(w_ref, o_ref):
    o_ref[...] = w_ref[...]


def kernel(x, W):
    del x
    S, D = W.shape
    blk = 2048
    return pl.pallas_call(
        _copy_block,
        grid=(S // blk,),
        in_specs=[pl.BlockSpec((blk, D), lambda i: (i, 0))],
        out_specs=pl.BlockSpec((blk, D), lambda i: (i, 0)),
        out_shape=jax.ShapeDtypeStruct((S, D), W.dtype),
        compiler_params=pltpu.CompilerParams(
            dimension_semantics=("parallel",),
        ),
    )(W)


# speedup vs baseline: 48.7236x; 1.0611x over previous
import jax
import jax.numpy as jnp
from jax.experimental import pallas as pl
from jax.experimental.pallas import tpu as pltpu


def _copy_block(w_ref, o_ref):
    o_ref[...] = w_ref[...]


def kernel(x, W):
    del x
    S, D = W.shape
    blk = 4096
    return pl.pallas_call(
        _copy_block,
        grid=(S // blk,),
        in_specs=[pl.BlockSpec((blk, D), lambda i: (i, 0))],
        out_specs=pl.BlockSpec((blk, D), lambda i: (i, 0)),
        out_shape=jax.ShapeDtypeStruct((S, D), W.dtype),
        compiler_params=pltpu.CompilerParams(
            dimension_semantics=("parallel",),
        ),
    )(W)
